# Initial kernel scaffold; baseline (speedup 1.0000x reference)
import jax
import jax.numpy as jnp
from jax.experimental import pallas as pl


def kernel(default, params, indices):
    raise NotImplementedError("write your pallas kernel here")


# speedup vs baseline: 7.0823x; 7.0823x over previous
import functools

import jax
import jax.numpy as jnp
from jax import lax
from jax.experimental import pallas as pl
from jax.experimental.pallas import tpu as pltpu
from jax.experimental.pallas import tpu_sc as plsc

_NC = 2
_NS = 16
_NW = _NC * _NS
_LANES = 16
_CHUNK = 8192


@functools.lru_cache(maxsize=None)
def _make_gather(total: int, p: int):
    chunk = _CHUNK
    per_w = total // _NW
    rounds = per_w // chunk
    mesh = plsc.VectorSubcoreMesh(
        core_axis_name="c", subcore_axis_name="s",
        num_cores=_NC, num_subcores=_NS)

    @functools.partial(
        pl.kernel,
        mesh=mesh,
        out_type=jax.ShapeDtypeStruct((total,), jnp.float32),
        scratch_types=[
            pltpu.VMEM((p,), jnp.float32),
            pltpu.VMEM((2 * chunk,), jnp.int32),
            pltpu.VMEM((chunk,), jnp.float32),
        ],
        compiler_params=pltpu.CompilerParams(needs_layout_passes=False),
    )
    def gather_kernel(params_hbm, raw_hbm, out_hbm, table_v, raw_v, res_v):
        wid = (lax.axis_index("s") * _NC + lax.axis_index("c")).astype(jnp.int32)
        pltpu.sync_copy(params_hbm, table_v)
        lane = lax.iota(jnp.int32, _LANES)

        def round_body(r, _):
            base = wid * per_w + r * chunk
            pltpu.sync_copy(raw_hbm.at[pl.ds(base * 2, 2 * chunk)], raw_v)

            @plsc.parallel_loop(0, chunk, _LANES, unroll=8)
            def j_body(o):
                ids2 = (lane + o) * 2
                lo = plsc.load_gather(raw_v, [ids2])
                hi = plsc.load_gather(raw_v, [ids2 + 1])
                vidx = lo | hi
                res_v[pl.ds(o, _LANES)] = plsc.load_gather(table_v, [vidx])
            pltpu.sync_copy(res_v, out_hbm.at[pl.ds(base, chunk)])
            return 0

        lax.fori_loop(0, rounds, round_body, 0)

    return gather_kernel


def kernel(default, params, indices):
    n, m = default.shape
    total = n * m
    raw = lax.bitcast_convert_type(indices, jnp.int32).reshape(2 * total)
    with jax.enable_x64(False):
        out_flat = _make_gather(total, params.shape[0])(params, raw)
    return out_flat.reshape(n, m)
